# baseline (speedup 1.0000x reference)
import functools

import jax
import jax.numpy as jnp
from jax import lax
from jax.experimental import pallas as pl
from jax.experimental.pallas import tpu as pltpu
from jax.experimental.pallas import tpu_sc as plsc

N = 10000
E = 320000
D_IN = 128
D_HID = 128
D_OUT = 64
DH = D_HID // 2

NC = 2
NS = 16
NW = NC * NS

CH = 125
K1 = E // NW // CH
K2 = E // NS // CH

NPAD = 10240
ROWS_PER_TILE = NPAD // NS

NBUF = 6
GAHEAD = 4
SLAG = 2

_mesh = plsc.VectorSubcoreMesh(core_axis_name="c", subcore_axis_name="s")
_sc_params = pltpu.CompilerParams(use_tc_tiling_on_sc=False)


@functools.partial(
    pl.kernel,
    out_type=jax.ShapeDtypeStruct((NC, NPAD), jnp.float32),
    mesh=_mesh,
    compiler_params=_sc_params,
    scratch_types=[
        pltpu.VMEM((K1, CH), jnp.int32),
        pltpu.VMEM((CH,), jnp.float32),
        pltpu.VMEM_SHARED((NPAD,), jnp.float32),
        pltpu.SemaphoreType.DMA,
    ],
)
def _deg(dstp, ones_hbm, zdeg_hbm, out, idx_v, ones_v, deg_sh, sem):
    c = lax.axis_index("c")
    s = lax.axis_index("s")
    pltpu.sync_copy(dstp.at[s, pl.ds(c * K1, K1)], idx_v)
    pltpu.sync_copy(ones_hbm, ones_v)
    pltpu.sync_copy(zdeg_hbm,
                    deg_sh.at[pl.ds(s * ROWS_PER_TILE, ROWS_PER_TILE)])
    plsc.subcore_barrier()
    hs = [None] * K1
    for k in range(K1):
        hs[k] = pltpu.async_copy(ones_v, deg_sh.at[idx_v.at[k]], sem, add=True)
        if k >= 16:
            hs[k - 16].wait()
    for k in range(K1 - 16, K1):
        hs[k].wait()
    plsc.subcore_barrier()
    pltpu.sync_copy(deg_sh.at[pl.ds(s * ROWS_PER_TILE, ROWS_PER_TILE)],
                    out.at[c, pl.ds(s * ROWS_PER_TILE, ROWS_PER_TILE)])


@functools.partial(
    pl.kernel,
    out_type=jax.ShapeDtypeStruct((NC, NPAD, DH), jnp.float32),
    mesh=_mesh,
    compiler_params=_sc_params,
    scratch_types=[
        pltpu.VMEM((K2, CH), jnp.int32),
        pltpu.VMEM((K2, CH), jnp.int32),
        pltpu.VMEM((NBUF, CH, DH), jnp.float32),
        pltpu.VMEM_SHARED((NPAD, DH), jnp.float32),
        pltpu.SemaphoreType.DMA,
        pltpu.SemaphoreType.DMA,
    ],
)
def _agg(srcp, dstp, xwph, zrows_hbm, out, src_v, dst_v, rows_v, agg_sh,
         gsem, ssem):
    c = lax.axis_index("c")
    s = lax.axis_index("s")
    pltpu.sync_copy(srcp.at[s], src_v)
    pltpu.sync_copy(dstp.at[s], dst_v)
    table = xwph.at[c]
    if True:
        lo = s * ROWS_PER_TILE

        @pl.when(s < NS - 1)
        def _():
            pltpu.sync_copy(table.at[pl.ds(lo, ROWS_PER_TILE)],
                            agg_sh.at[pl.ds(lo, ROWS_PER_TILE)])

        @pl.when(s == NS - 1)
        def _():
            pltpu.sync_copy(table.at[pl.ds(N - 400, 400)],
                            agg_sh.at[pl.ds(N - 400, 400)])
            pltpu.sync_copy(zrows_hbm, agg_sh.at[pl.ds(N, NPAD - N)])
    plsc.subcore_barrier()
    gh = [None] * K2
    sh = [None] * K2
    for k in range(GAHEAD):
        gh[k] = pltpu.async_copy(table.at[src_v.at[k]], rows_v.at[k % NBUF], gsem)
    for k in range(K2):
        b = k % NBUF
        gh[k].wait()
        sh[k] = pltpu.async_copy(rows_v.at[b], agg_sh.at[dst_v.at[k]], ssem,
                                 add=True)
        if k >= SLAG:
            sh[k - SLAG].wait()
        nk = k + GAHEAD
        if nk < K2:
            gh[nk] = pltpu.async_copy(table.at[src_v.at[nk]],
                                      rows_v.at[nk % NBUF], gsem)
    for k in range(max(0, K2 - SLAG), K2):
        sh[k].wait()
    plsc.subcore_barrier()
    pltpu.sync_copy(agg_sh.at[pl.ds(s * ROWS_PER_TILE, ROWS_PER_TILE)],
                    out.at[c, pl.ds(s * ROWS_PER_TILE, ROWS_PER_TILE)])


_RB = 2000


_CRB = _RB // 2
_HALF = N // 2


def _prep_body(xa_ref, xb_ref, w1_ref, dg_ref, xwph_ref, dis_ref):
    dis_a = lax.rsqrt(dg_ref[0][:, 0:1] + dg_ref[1][:, 0:1] + 1.0)
    dis_b = lax.rsqrt(dg_ref[0][:, 1:2] + dg_ref[1][:, 1:2] + 1.0)
    xwa = jnp.dot(xa_ref[...], w1_ref[...], preferred_element_type=jnp.float32) * dis_a
    xwb = jnp.dot(xb_ref[...], w1_ref[...], preferred_element_type=jnp.float32) * dis_b
    xwph_ref[0] = jnp.concatenate([xwa[:, :DH], xwb[:, :DH]], axis=1)
    xwph_ref[1] = jnp.concatenate([xwa[:, DH:], xwb[:, DH:]], axis=1)
    dis_ref[...] = jnp.concatenate([dis_a, dis_b], axis=1)


_prep = pl.pallas_call(
    _prep_body,
    grid=(_HALF // _CRB,),
    in_specs=[
        pl.BlockSpec((_CRB, D_IN), lambda i: (i, 0)),
        pl.BlockSpec((_CRB, D_IN), lambda i: (i + _HALF // _CRB, 0)),
        pl.BlockSpec((D_IN, D_HID), lambda i: (0, 0)),
        pl.BlockSpec((NC, _CRB, 2), lambda i: (0, i, 0)),
    ],
    out_specs=[
        pl.BlockSpec((NC, _CRB, D_HID), lambda i: (0, i, 0)),
        pl.BlockSpec((_CRB, 2), lambda i: (i, 0)),
    ],
    out_shape=[
        jax.ShapeDtypeStruct((NC, _HALF, D_HID), jnp.float32),
        jax.ShapeDtypeStruct((_HALF, 2), jnp.float32),
    ],
)


def _out_body(aggc_ref, dis2_ref, b1_ref, w2_ref, b2_ref, y_ref):
    d_a = jnp.broadcast_to(dis2_ref[:, 0:1], (_CRB, DH))
    d_b = jnp.broadcast_to(dis2_ref[:, 1:2], (_CRB, DH))
    disc = jnp.concatenate([d_a, d_b], axis=1)
    b1lo = jnp.concatenate([b1_ref[:, :DH], b1_ref[:, :DH]], axis=1)
    b1hi = jnp.concatenate([b1_ref[:, DH:], b1_ref[:, DH:]], axis=1)
    h0 = jnp.maximum(aggc_ref[0] * disc + b1lo, 0.0)
    h1 = jnp.maximum(aggc_ref[1] * disc + b1hi, 0.0)
    w2lo, w2hi = w2_ref[:DH], w2_ref[DH:]
    y_a = (jnp.dot(h0[:, :DH], w2lo, preferred_element_type=jnp.float32)
           + jnp.dot(h1[:, :DH], w2hi, preferred_element_type=jnp.float32)
           + b2_ref[...])
    y_b = (jnp.dot(h0[:, DH:], w2lo, preferred_element_type=jnp.float32)
           + jnp.dot(h1[:, DH:], w2hi, preferred_element_type=jnp.float32)
           + b2_ref[...])
    y_ref[0] = y_a
    y_ref[1] = y_b


_out = pl.pallas_call(
    _out_body,
    grid=(_HALF // _CRB,),
    in_specs=[
        pl.BlockSpec((NC, _CRB, D_HID), lambda i: (0, i, 0)),
        pl.BlockSpec((_CRB, 2), lambda i: (i, 0)),
        pl.BlockSpec((1, D_HID), lambda i: (0, 0)),
        pl.BlockSpec((D_HID, D_OUT), lambda i: (0, 0)),
        pl.BlockSpec((1, D_OUT), lambda i: (0, 0)),
    ],
    out_specs=pl.BlockSpec((2, _CRB, D_OUT), lambda i: (0, i, 0)),
    out_shape=jax.ShapeDtypeStruct((2, _HALF, D_OUT), jnp.float32),
)


def kernel(x, edge_index, W1, b1, W2, b2):
    src_t = (edge_index[0] % _HALF) * 2 + edge_index[0] // _HALF
    dst_t = (edge_index[1] % _HALF) * 2 + edge_index[1] // _HALF
    srcp2 = src_t.reshape(NS, K2, CH)
    dstp2 = dst_t.reshape(NS, K2, CH)

    ones = jnp.ones((CH,), jnp.float32)
    zdeg = jnp.zeros((ROWS_PER_TILE,), jnp.float32)
    zrows = jnp.zeros((NPAD - N, DH), jnp.float32)

    degp = _deg(dstp2, ones, zdeg)
    degp2 = degp[:, :N].reshape(NC, _HALF, 2)
    xwphc, dis2 = _prep(x, x, W1, degp2)
    xwph = xwphc.reshape(NC, N, DH)
    aggh = _agg(srcp2, dstp2, xwph, zrows)
    aggc = aggh.reshape(NC, NPAD // 2, 2 * DH)
    y2 = _out(aggc, dis2, b1.reshape(1, D_HID), W2, b2.reshape(1, D_OUT))
    return y2.reshape(N, D_OUT)


# speedup vs baseline: 1.1211x; 1.1211x over previous
import functools

import jax
import jax.numpy as jnp
from jax import lax
from jax.experimental import pallas as pl
from jax.experimental.pallas import tpu as pltpu
from jax.experimental.pallas import tpu_sc as plsc

N = 10000
E = 320000
D_IN = 128
D_HID = 128
D_OUT = 64
DH = D_HID // 2

NC = 2
NS = 16
NW = NC * NS

CH = 125
K1 = E // NW // CH
K2 = E // NS // CH

NPAD = 10240
ROWS_PER_TILE = NPAD // NS

NBUF = 6
GAHEAD = 4
SLAG = 2

_mesh = plsc.VectorSubcoreMesh(core_axis_name="c", subcore_axis_name="s")
_sc_params = pltpu.CompilerParams(use_tc_tiling_on_sc=False)


@functools.partial(
    pl.kernel,
    out_type=jax.ShapeDtypeStruct((NC, NPAD), jnp.float32),
    mesh=_mesh,
    compiler_params=_sc_params,
    scratch_types=[
        pltpu.VMEM((K1, CH), jnp.int32),
        pltpu.VMEM((CH,), jnp.float32),
        pltpu.VMEM_SHARED((NPAD,), jnp.float32),
        pltpu.SemaphoreType.DMA,
    ],
)
def _deg(dstp, ones_hbm, zdeg_hbm, out, idx_v, ones_v, deg_sh, sem):
    c = lax.axis_index("c")
    s = lax.axis_index("s")
    pltpu.sync_copy(dstp.at[s, pl.ds(c * K1, K1)], idx_v)
    pltpu.sync_copy(ones_hbm, ones_v)
    pltpu.sync_copy(zdeg_hbm,
                    deg_sh.at[pl.ds(s * ROWS_PER_TILE, ROWS_PER_TILE)])
    plsc.subcore_barrier()
    hs = [None] * K1
    for k in range(K1):
        hs[k] = pltpu.async_copy(ones_v, deg_sh.at[idx_v.at[k]], sem, add=True)
        if k >= 16:
            hs[k - 16].wait()
    for k in range(K1 - 16, K1):
        hs[k].wait()
    plsc.subcore_barrier()
    pltpu.sync_copy(deg_sh.at[pl.ds(s * ROWS_PER_TILE, ROWS_PER_TILE)],
                    out.at[c, pl.ds(s * ROWS_PER_TILE, ROWS_PER_TILE)])


@functools.partial(
    pl.kernel,
    out_type=jax.ShapeDtypeStruct((NC, NPAD, DH), jnp.float32),
    mesh=_mesh,
    compiler_params=_sc_params,
    scratch_types=[
        pltpu.VMEM((K2, CH), jnp.int32),
        pltpu.VMEM((K2, CH), jnp.int32),
        pltpu.VMEM((NBUF, CH, DH), jnp.float32),
        pltpu.VMEM_SHARED((NPAD, DH), jnp.float32),
        pltpu.SemaphoreType.DMA,
        pltpu.SemaphoreType.DMA,
    ],
)
def _agg(srcp, dstp, xwph, zrows_hbm, out, src_v, dst_v, rows_v, agg_sh,
         gsem, ssem):
    c = lax.axis_index("c")
    s = lax.axis_index("s")
    pltpu.sync_copy(srcp.at[s], src_v)
    pltpu.sync_copy(dstp.at[s], dst_v)
    table = xwph.at[c]
    if True:
        lo = s * ROWS_PER_TILE

        @pl.when(s < NS - 1)
        def _():
            pltpu.sync_copy(table.at[pl.ds(lo, ROWS_PER_TILE)],
                            agg_sh.at[pl.ds(lo, ROWS_PER_TILE)])

        @pl.when(s == NS - 1)
        def _():
            pltpu.sync_copy(table.at[pl.ds(N - 400, 400)],
                            agg_sh.at[pl.ds(N - 400, 400)])
            pltpu.sync_copy(zrows_hbm, agg_sh.at[pl.ds(N, NPAD - N)])
    plsc.subcore_barrier()
    gh = [None] * K2
    sh = [None] * K2
    for k in range(GAHEAD):
        gh[k] = pltpu.async_copy(table.at[src_v.at[k]], rows_v.at[k % NBUF], gsem)
    for k in range(K2):
        b = k % NBUF
        gh[k].wait()
        sh[k] = pltpu.async_copy(rows_v.at[b], agg_sh.at[dst_v.at[k]], ssem,
                                 add=True)
        if k >= SLAG:
            sh[k - SLAG].wait()
        nk = k + GAHEAD
        if nk < K2:
            gh[nk] = pltpu.async_copy(table.at[src_v.at[nk]],
                                      rows_v.at[nk % NBUF], gsem)
    for k in range(max(0, K2 - SLAG), K2):
        sh[k].wait()
    plsc.subcore_barrier()
    pltpu.sync_copy(agg_sh.at[pl.ds(s * ROWS_PER_TILE, ROWS_PER_TILE)],
                    out.at[c, pl.ds(s * ROWS_PER_TILE, ROWS_PER_TILE)])


_RB = 2000


def _prep_body(x_ref, w1_ref, degp_ref, xwph_ref, dis_ref):
    deg = degp_ref[0] + degp_ref[1] + 1.0
    dis = lax.rsqrt(deg)
    xw = jnp.dot(x_ref[...], w1_ref[...], preferred_element_type=jnp.float32)
    xwp = xw * dis
    xwph_ref[0] = xwp[:, :DH]
    xwph_ref[1] = xwp[:, DH:]
    dis_ref[...] = dis


_prep = pl.pallas_call(
    _prep_body,
    grid=(N // _RB,),
    in_specs=[
        pl.BlockSpec((_RB, D_IN), lambda i: (i, 0)),
        pl.BlockSpec((D_IN, D_HID), lambda i: (0, 0)),
        pl.BlockSpec((NC, _RB, 1), lambda i: (0, i, 0)),
    ],
    out_specs=[
        pl.BlockSpec((NC, _RB, DH), lambda i: (0, i, 0)),
        pl.BlockSpec((_RB, 1), lambda i: (i, 0)),
    ],
    out_shape=[
        jax.ShapeDtypeStruct((NC, N, DH), jnp.float32),
        jax.ShapeDtypeStruct((N, 1), jnp.float32),
    ],
)


_CRB = _RB // 2


def _out_body(aggc_ref, dis2_ref, b1_ref, w2_ref, b2c_ref, y_ref):
    d_even = jnp.broadcast_to(dis2_ref[:, 0:1], (_CRB, DH))
    d_odd = jnp.broadcast_to(dis2_ref[:, 1:2], (_CRB, DH))
    disc = jnp.concatenate([d_even, d_odd], axis=1)
    b1lo = jnp.concatenate([b1_ref[:, :DH], b1_ref[:, :DH]], axis=1)
    b1hi = jnp.concatenate([b1_ref[:, DH:], b1_ref[:, DH:]], axis=1)
    h0 = jnp.maximum(aggc_ref[0] * disc + b1lo, 0.0)
    h1 = jnp.maximum(aggc_ref[1] * disc + b1hi, 0.0)
    w2lo, w2hi = w2_ref[:DH], w2_ref[DH:]
    y_even = (jnp.dot(h0[:, :DH], w2lo, preferred_element_type=jnp.float32)
              + jnp.dot(h1[:, :DH], w2hi, preferred_element_type=jnp.float32))
    y_odd = (jnp.dot(h0[:, DH:], w2lo, preferred_element_type=jnp.float32)
             + jnp.dot(h1[:, DH:], w2hi, preferred_element_type=jnp.float32))
    y_ref[...] = jnp.concatenate([y_even, y_odd], axis=1) + b2c_ref[...]


_out = pl.pallas_call(
    _out_body,
    grid=(N // _RB,),
    in_specs=[
        pl.BlockSpec((NC, _CRB, D_HID), lambda i: (0, i, 0)),
        pl.BlockSpec((_CRB, 2), lambda i: (i, 0)),
        pl.BlockSpec((1, D_HID), lambda i: (0, 0)),
        pl.BlockSpec((D_HID, D_OUT), lambda i: (0, 0)),
        pl.BlockSpec((1, 2 * D_OUT), lambda i: (0, 0)),
    ],
    out_specs=pl.BlockSpec((_CRB, 2 * D_OUT), lambda i: (i, 0)),
    out_shape=jax.ShapeDtypeStruct((N // 2, 2 * D_OUT), jnp.float32),
)


def kernel(x, edge_index, W1, b1, W2, b2):
    srcp2 = edge_index[0].reshape(NS, K2, CH)
    dstp2 = edge_index[1].reshape(NS, K2, CH)

    ones = jnp.ones((CH,), jnp.float32)
    zdeg = jnp.zeros((ROWS_PER_TILE,), jnp.float32)
    zrows = jnp.zeros((NPAD - N, DH), jnp.float32)

    degp = _deg(dstp2, ones, zdeg)
    degp_col = degp[:, :N, None]
    xwph, dis = _prep(x, W1, degp_col)
    aggh = _agg(srcp2, dstp2, xwph, zrows)
    aggc = aggh.reshape(NC, NPAD // 2, 2 * DH)
    dis2 = dis.reshape(N // 2, 2)
    b2c = jnp.concatenate([b2, b2]).reshape(1, 2 * D_OUT)
    yc = _out(aggc, dis2, b1.reshape(1, D_HID), W2, b2c)
    return yc.reshape(N, D_OUT)
